# Initial kernel scaffold; baseline (speedup 1.0000x reference)
import jax
import jax.numpy as jnp
from jax.experimental import pallas as pl


def kernel(table, seq_len, past_key_values_length):
    raise NotImplementedError("write your pallas kernel here")


# speedup vs baseline: 83.5368x; 83.5368x over previous
import math

import jax
import jax.numpy as jnp
from jax.experimental import pallas as pl
from jax.experimental.pallas import tpu as pltpu

_NB = 32
_NH = 16
_S = 2048
_W = 4224
_THR = [math.ceil(16 * 8 ** (t / 16)) for t in range(1, 16)]


def _body(delta_ref, table_ref, out_ref, rolled_ref):
    rb = pl.program_id(1)

    @pl.when(rb == 0)
    def _build():
        delta = delta_ref[0, 0]
        c = jax.lax.broadcasted_iota(jnp.int32, (1, _W), 1)
        n = jnp.abs(c + (delta - _S))
        large = jnp.full((1, _W), 16, jnp.int32)
        for t in _THR:
            large += (n >= t).astype(jnp.int32)
        bucket = jnp.where(n < 16, n, large)
        vals = jnp.zeros((1, _W), jnp.float32)
        for b in range(_NB):
            vals = jnp.where(bucket == b, table_ref[0, 0, b], vals)
        x = jnp.broadcast_to(vals, (128, _W))
        m = jax.lax.broadcasted_iota(jnp.int32, (128, _W), 0)
        for t in range(7):
            sh = 1 << t
            x = jnp.where((m & sh) != 0, pltpu.roll(x, sh, 1), x)
        rolled_ref[...] = x

    start = _S - 128 * rb
    out_ref[0, 0, :, :] = rolled_ref[:, pl.ds(start, _S)]


def kernel(table, seq_len, past_key_values_length):
    delta = (jnp.asarray(seq_len).astype(jnp.int32) - _S
             - jnp.asarray(past_key_values_length).astype(jnp.int32)).reshape(1, 1)
    tpad = jnp.zeros((_NH, 1, 128), jnp.float32).at[:, 0, :_NB].set(table.astype(jnp.float32).T)
    return pl.pallas_call(
        _body,
        grid=(_NH, _S // 128),
        in_specs=[
            pl.BlockSpec((1, 1), lambda h, rb: (0, 0), memory_space=pltpu.SMEM),
            pl.BlockSpec((1, 1, 128), lambda h, rb: (h, 0, 0), memory_space=pltpu.SMEM),
        ],
        out_specs=pl.BlockSpec((1, 1, 128, _S), lambda h, rb: (0, h, rb, 0)),
        out_shape=jax.ShapeDtypeStruct((1, _NH, _S, _S), jnp.float32),
        scratch_shapes=[pltpu.VMEM((128, _W), jnp.float32)],
        compiler_params=pltpu.CompilerParams(
            dimension_semantics=("arbitrary", "arbitrary"),
        ),
    )(delta, tpad)


# speedup vs baseline: 184.0117x; 2.2028x over previous
import math

import jax
import jax.numpy as jnp
from jax.experimental import pallas as pl
from jax.experimental.pallas import tpu as pltpu

_NB = 32
_NH = 16
_S = 2048
_W = 4224
_THR = [math.ceil(16 * 8 ** (t / 16)) for t in range(1, 16)]


def _build_rolled(delta, table_ref, h):
    c = jax.lax.broadcasted_iota(jnp.int32, (1, _W), 1)
    n = jnp.abs(c + (delta - _S))
    large = jnp.full((1, _W), 16, jnp.int32)
    for t in _THR:
        large += (n >= t).astype(jnp.int32)
    bucket = jnp.where(n < 16, n, large)
    vals = jnp.zeros((1, _W), jnp.float32)
    for b in range(_NB):
        vals = jnp.where(bucket == b, table_ref[0, 0, b], vals)
    x = jnp.broadcast_to(vals, (128, _W))
    m = jax.lax.broadcasted_iota(jnp.int32, (128, _W), 0)
    for t in range(7):
        sh = 1 << t
        x = jnp.where((m & sh) != 0, pltpu.roll(x, sh, 1), x)
    return x


def _copies(rolled_ref, out_hbm, sem, bi, h):
    for rb in range(_S // 128):
        yield pltpu.make_async_copy(
            rolled_ref.at[bi, :, pl.ds(_S - 128 * rb, _S)],
            out_hbm.at[0, h, pl.ds(128 * rb, 128), :],
            sem,
        )


def _body(delta_ref, table_ref, out_hbm, rolled_ref, sem0, sem1):
    h = pl.program_id(0)
    even = (h % 2) == 0

    def run(bi, sem):
        @pl.when(h >= 2)
        def _drain_prev():
            for cp in _copies(rolled_ref, out_hbm, sem, bi, h - 2):
                cp.wait()

        rolled = _build_rolled(delta_ref[0, 0], table_ref, h)
        rolled_ref[bi] = rolled
        for cp in _copies(rolled_ref, out_hbm, sem, bi, h):
            cp.start()

    @pl.when(even)
    def _even():
        run(0, sem0)

    @pl.when(jnp.logical_not(even))
    def _odd():
        run(1, sem1)

    @pl.when(h == _NH - 1)
    def _final_drain():
        for cp in _copies(rolled_ref, out_hbm, sem0, 0, h - 1):
            cp.wait()
        for cp in _copies(rolled_ref, out_hbm, sem1, 1, h):
            cp.wait()


def kernel(table, seq_len, past_key_values_length):
    delta = (jnp.asarray(seq_len).astype(jnp.int32) - _S
             - jnp.asarray(past_key_values_length).astype(jnp.int32)).reshape(1, 1)
    tpad = jnp.zeros((_NH, 1, 128), jnp.float32).at[:, 0, :_NB].set(table.astype(jnp.float32).T)
    return pl.pallas_call(
        _body,
        grid=(_NH,),
        in_specs=[
            pl.BlockSpec((1, 1), lambda h: (0, 0), memory_space=pltpu.SMEM),
            pl.BlockSpec((1, 1, 128), lambda h: (h, 0, 0), memory_space=pltpu.SMEM),
        ],
        out_specs=pl.BlockSpec(memory_space=pl.ANY),
        out_shape=jax.ShapeDtypeStruct((1, _NH, _S, _S), jnp.float32),
        scratch_shapes=[
            pltpu.VMEM((2, 128, _W), jnp.float32),
            pltpu.SemaphoreType.DMA,
            pltpu.SemaphoreType.DMA,
        ],
        compiler_params=pltpu.CompilerParams(
            dimension_semantics=("arbitrary",),
        ),
    )(delta, tpad)
